# baseline (speedup 1.0000x reference)
import functools

import jax
import jax.numpy as jnp
from jax import lax
from jax.experimental import pallas as pl
from jax.experimental.pallas import tpu as pltpu
from jax.experimental.pallas import tpu_sc as plsc

NC, NS, L = 2, 16, 16
NW = NC * NS

N_NODES = 100000
N_PAD = 102400
N_PER_W = N_PAD // NW
N_EDGES = 6400000
E_PER_W = N_EDGES // NW
EROWS = 500
ECOLS_W = 400
RCHUNK = 20
N_CHUNKS = EROWS // RCHUNK
GROWS = 128
NG = N_PER_W // GROWS
HSUB = 640
NHS = N_PER_W // HSUB
MAX_DEG = 256
EMB = 64

_mesh = dict(core_axis_name="c", subcore_axis_name="s")


@functools.partial(
    pl.kernel,
    out_type=jax.ShapeDtypeStruct((NW, N_PAD), jnp.int32),
    mesh=plsc.VectorSubcoreMesh(**_mesh),
    scratch_types=[
        pltpu.VMEM((RCHUNK, ECOLS_W), jnp.int32),
        pltpu.VMEM((RCHUNK, ECOLS_W), jnp.int32),
        pltpu.VMEM((N_PAD,), jnp.int32),
        pltpu.SemaphoreType.DMA,
        pltpu.SemaphoreType.DMA,
    ],
    compiler_params=pltpu.CompilerParams(needs_layout_passes=False,
                                         use_tc_tiling_on_sc=False),
)
def _hist_kernel(edges_hbm, hists_hbm, ebuf0, ebuf1, hist, sem0, sem1):
    wid = lax.axis_index("s") * NC + lax.axis_index("c")
    zeros = jnp.zeros((L,), jnp.int32)
    ones = jnp.ones((L,), jnp.int32)

    with jax.named_scope("ph_zero"):
        @plsc.parallel_loop(0, N_PAD, step=L, unroll=8)
        def _(i):
            hist[pl.ds(i, L)] = zeros

    cbase = wid * ECOLS_W
    bufs = (ebuf0, ebuf1)
    sems = (sem0, sem1)
    desc = pltpu.async_copy(
        edges_hbm.at[pl.ds(0, RCHUNK), pl.ds(cbase, ECOLS_W)], ebuf0, sem0)
    for c in range(N_CHUNKS):
        nxt = None
        if c + 1 < N_CHUNKS:
            b = (c + 1) % 2
            nxt = pltpu.async_copy(
                edges_hbm.at[pl.ds((c + 1) * RCHUNK, RCHUNK),
                             pl.ds(cbase, ECOLS_W)],
                bufs[b], sems[b])
        with jax.named_scope("ph_wait"):
            desc.wait()
        buf = bufs[c % 2]

        with jax.named_scope("ph_scat"):
            @plsc.parallel_loop(0, RCHUNK, unroll=2)
            def _(r):
                for cc in range(ECOLS_W // L):
                    ids = buf[r, pl.ds(cc * L, L)]
                    plsc.addupdate_scatter(hist, [ids], ones)

        desc = nxt

    with jax.named_scope("ph_wb"):
        pltpu.sync_copy(hist, hists_hbm.at[wid])


@functools.partial(
    pl.kernel,
    out_type=jax.ShapeDtypeStruct((N_PAD, EMB), jnp.float32),
    mesh=plsc.VectorSubcoreMesh(**_mesh),
    scratch_types=[
        pltpu.VMEM((NW, HSUB), jnp.int32),
        pltpu.VMEM((N_PER_W,), jnp.int32),
        pltpu.VMEM((GROWS, EMB), jnp.float32),
        pltpu.VMEM((GROWS, EMB), jnp.float32),
        pltpu.SemaphoreType.DMA,
        pltpu.SemaphoreType.DMA,
    ],
    compiler_params=pltpu.CompilerParams(needs_layout_passes=False,
                                         use_tc_tiling_on_sc=False),
)
def _lookup_kernel(hists_hbm, table_hbm, out_hbm, hsl, didx, rows0, rows1,
                   gsem0, gsem1):
    wid = lax.axis_index("s") * NC + lax.axis_index("c")
    nbase = wid * N_PER_W

    for s in range(NHS):
        pltpu.sync_copy(hists_hbm.at[:, pl.ds(nbase + s * HSUB, HSUB)], hsl)

        @plsc.parallel_loop(0, HSUB, step=L, unroll=2)
        def _(i):
            acc = hsl[0, pl.ds(i, L)]
            for t in range(1, NW):
                acc = acc + hsl[t, pl.ds(i, L)]
            acc = jnp.minimum(acc, MAX_DEG)
            didx[pl.ds(s * HSUB + i, L)] = acc

    gsems = (gsem0, gsem1)
    rbufs = (rows0, rows1)
    desc = pltpu.async_copy(
        table_hbm.at[didx.at[pl.ds(0, GROWS)]], rows0, gsem0)
    for j in range(NG):
        nxt = None
        if j + 1 < NG:
            b = (j + 1) % 2
            nxt = pltpu.async_copy(
                table_hbm.at[didx.at[pl.ds((j + 1) * GROWS, GROWS)]],
                rbufs[b], gsems[b])
        desc.wait()
        pltpu.sync_copy(rbufs[j % 2],
                        out_hbm.at[pl.ds(nbase + j * GROWS, GROWS)])
        desc = nxt


def kernel(edge_index, num_nodes, table):
    edges = edge_index[0].reshape(EROWS, NW * ECOLS_W)
    hists = _hist_kernel(edges)
    out = _lookup_kernel(hists, table)
    return out[:N_NODES]


# speedup vs baseline: 1.0012x; 1.0012x over previous
import functools

import jax
import jax.numpy as jnp
from jax import lax
from jax.experimental import pallas as pl
from jax.experimental.pallas import tpu as pltpu
from jax.experimental.pallas import tpu_sc as plsc

NC, NS, L = 2, 16, 16
NW = NC * NS

N_NODES = 100000
N_PAD = 102400
N_PER_W = N_PAD // NW
N_EDGES = 6400000
E_PER_W = N_EDGES // NW
EROWS = 500
ECOLS_W = 400
RCHUNK = 20
N_CHUNKS = EROWS // RCHUNK
GROWS = 128
NG = N_PER_W // GROWS
HSUB = 640
NHS = N_PER_W // HSUB
MAX_DEG = 256
EMB = 64

_mesh = dict(core_axis_name="c", subcore_axis_name="s")


@functools.partial(
    pl.kernel,
    out_type=jax.ShapeDtypeStruct((NW, N_PAD), jnp.int32),
    mesh=plsc.VectorSubcoreMesh(**_mesh),
    scratch_types=[
        pltpu.VMEM((RCHUNK, ECOLS_W), jnp.int32),
        pltpu.VMEM((RCHUNK, ECOLS_W), jnp.int32),
        pltpu.VMEM((N_PAD,), jnp.int32),
        pltpu.SemaphoreType.DMA,
        pltpu.SemaphoreType.DMA,
    ],
    compiler_params=pltpu.CompilerParams(needs_layout_passes=False,
                                         use_tc_tiling_on_sc=False),
)
def _hist_kernel(edges_hbm, hists_hbm, ebuf0, ebuf1, hist, sem0, sem1):
    wid = lax.axis_index("s") * NC + lax.axis_index("c")
    zeros = jnp.zeros((L,), jnp.int32)
    ones = jnp.ones((L,), jnp.int32)

    with jax.named_scope("ph_zero"):
        @plsc.parallel_loop(0, N_PAD, step=L, unroll=8)
        def _(i):
            hist[pl.ds(i, L)] = zeros

    cbase = wid * ECOLS_W
    bufs = (ebuf0, ebuf1)
    sems = (sem0, sem1)
    with jax.named_scope("ph_iss"):
        desc = pltpu.async_copy(
            edges_hbm.at[pl.ds(0, RCHUNK), pl.ds(cbase, ECOLS_W)], ebuf0, sem0)
    for c in range(N_CHUNKS):
        nxt = None
        if c + 1 < N_CHUNKS:
            b = (c + 1) % 2
            with jax.named_scope("ph_iss"):
                nxt = pltpu.async_copy(
                    edges_hbm.at[pl.ds((c + 1) * RCHUNK, RCHUNK),
                                 pl.ds(cbase, ECOLS_W)],
                    bufs[b], sems[b])
        with jax.named_scope("ph_wait"):
            desc.wait()
        buf = bufs[c % 2]

        with jax.named_scope("ph_scat"):
            @plsc.parallel_loop(0, RCHUNK, unroll=2)
            def _(r):
                for cc in range(ECOLS_W // L):
                    ids = buf[r, pl.ds(cc * L, L)]
                    plsc.addupdate_scatter(hist, [ids], ones)

        desc = nxt

    with jax.named_scope("ph_wb"):
        pltpu.sync_copy(hist, hists_hbm.at[wid])


@functools.partial(
    pl.kernel,
    out_type=jax.ShapeDtypeStruct((N_PAD, EMB), jnp.float32),
    mesh=plsc.VectorSubcoreMesh(**_mesh),
    scratch_types=[
        pltpu.VMEM((NW, HSUB), jnp.int32),
        pltpu.VMEM((N_PER_W,), jnp.int32),
        pltpu.VMEM((GROWS, EMB), jnp.float32),
        pltpu.VMEM((GROWS, EMB), jnp.float32),
        pltpu.SemaphoreType.DMA,
        pltpu.SemaphoreType.DMA,
    ],
    compiler_params=pltpu.CompilerParams(needs_layout_passes=False,
                                         use_tc_tiling_on_sc=False),
)
def _lookup_kernel(hists_hbm, table_hbm, out_hbm, hsl, didx, rows0, rows1,
                   gsem0, gsem1):
    wid = lax.axis_index("s") * NC + lax.axis_index("c")
    nbase = wid * N_PER_W

    for s in range(NHS):
        pltpu.sync_copy(hists_hbm.at[:, pl.ds(nbase + s * HSUB, HSUB)], hsl)

        @plsc.parallel_loop(0, HSUB, step=L, unroll=2)
        def _(i):
            acc = hsl[0, pl.ds(i, L)]
            for t in range(1, NW):
                acc = acc + hsl[t, pl.ds(i, L)]
            acc = jnp.minimum(acc, MAX_DEG)
            didx[pl.ds(s * HSUB + i, L)] = acc

    gsems = (gsem0, gsem1)
    rbufs = (rows0, rows1)
    desc = pltpu.async_copy(
        table_hbm.at[didx.at[pl.ds(0, GROWS)]], rows0, gsem0)
    for j in range(NG):
        nxt = None
        if j + 1 < NG:
            b = (j + 1) % 2
            nxt = pltpu.async_copy(
                table_hbm.at[didx.at[pl.ds((j + 1) * GROWS, GROWS)]],
                rbufs[b], gsems[b])
        desc.wait()
        pltpu.sync_copy(rbufs[j % 2],
                        out_hbm.at[pl.ds(nbase + j * GROWS, GROWS)])
        desc = nxt


def kernel(edge_index, num_nodes, table):
    edges = edge_index[0].reshape(EROWS, NW * ECOLS_W)
    hists = _hist_kernel(edges)
    out = _lookup_kernel(hists, table)
    return out[:N_NODES]


# speedup vs baseline: 2.4376x; 2.4346x over previous
import functools

import jax
import jax.numpy as jnp
from jax import lax
from jax.experimental import pallas as pl
from jax.experimental.pallas import tpu as pltpu
from jax.experimental.pallas import tpu_sc as plsc

NC, NS, L = 2, 16, 16
NW = NC * NS

N_NODES = 100000
N_PAD = 102400
N_PER_W = N_PAD // NW
N_EDGES = 6400000
E_PER_W = N_EDGES // NW
EROWS = 500
ECOLS_W = 400
RCHUNK = 20
N_CHUNKS = EROWS // RCHUNK
GROWS = 128
NG = N_PER_W // GROWS
HSUB = 640
NHS = N_PER_W // HSUB
MAX_DEG = 256
EMB = 64

_mesh = dict(core_axis_name="c", subcore_axis_name="s")


@functools.partial(
    pl.kernel,
    out_type=jax.ShapeDtypeStruct((NW, N_PAD), jnp.int32),
    mesh=plsc.VectorSubcoreMesh(**_mesh),
    scratch_types=[
        pltpu.VMEM((RCHUNK, ECOLS_W), jnp.int32),
        pltpu.VMEM((RCHUNK, ECOLS_W), jnp.int32),
        pltpu.VMEM((N_PAD,), jnp.int32),
        pltpu.SemaphoreType.DMA,
        pltpu.SemaphoreType.DMA,
    ],
    compiler_params=pltpu.CompilerParams(needs_layout_passes=False,
                                         use_tc_tiling_on_sc=False),
)
def _hist_kernel(edges_hbm, hists_hbm, ebuf0, ebuf1, hist, sem0, sem1):
    wid = lax.axis_index("s") * NC + lax.axis_index("c")
    zeros = jnp.zeros((L,), jnp.int32)
    ones = jnp.ones((L,), jnp.int32)

    with jax.named_scope("ph_zero"):
        @plsc.parallel_loop(0, N_PAD, step=L, unroll=8)
        def _(i):
            hist[pl.ds(i, L)] = zeros

    cbase = wid * ECOLS_W
    bufs = (ebuf0, ebuf1)
    sems = (sem0, sem1)
    with jax.named_scope("ph_iss"):
        desc = pltpu.async_copy(
            edges_hbm.at[pl.ds(0, RCHUNK), pl.ds(cbase, ECOLS_W)], ebuf0, sem0)
    for c in range(N_CHUNKS):
        nxt = None
        if c + 1 < N_CHUNKS:
            b = (c + 1) % 2
            with jax.named_scope("ph_iss"):
                nxt = pltpu.async_copy(
                    edges_hbm.at[pl.ds((c + 1) * RCHUNK, RCHUNK),
                                 pl.ds(cbase, ECOLS_W)],
                    bufs[b], sems[b])
        with jax.named_scope("ph_wait"):
            desc.wait()
        buf = bufs[c % 2]

        with jax.named_scope("ph_scat"):
            @plsc.parallel_loop(0, RCHUNK, unroll=2)
            def _(r):
                for cc in range(ECOLS_W // L):
                    ids = buf[r, pl.ds(cc * L, L)]
                    plsc.addupdate_scatter(hist, [ids], ones)

        desc = nxt

    with jax.named_scope("ph_wb"):
        pltpu.sync_copy(hist, hists_hbm.at[wid])


@functools.partial(
    pl.kernel,
    out_type=jax.ShapeDtypeStruct((N_PAD, EMB), jnp.float32),
    mesh=plsc.VectorSubcoreMesh(**_mesh),
    scratch_types=[
        pltpu.VMEM((NW, HSUB), jnp.int32),
        pltpu.VMEM((N_PER_W,), jnp.int32),
        pltpu.VMEM((GROWS, EMB), jnp.float32),
        pltpu.VMEM((GROWS, EMB), jnp.float32),
        pltpu.VMEM_SHARED((MAX_DEG + 1, EMB), jnp.float32),
        pltpu.SemaphoreType.DMA,
        pltpu.SemaphoreType.DMA,
    ],
    compiler_params=pltpu.CompilerParams(needs_layout_passes=False,
                                         use_tc_tiling_on_sc=False),
)
def _lookup_kernel(hists_hbm, table_hbm, out_hbm, hsl, didx, rows0, rows1,
                   table_sp, gsem0, gsem1):
    wid = lax.axis_index("s") * NC + lax.axis_index("c")
    nbase = wid * N_PER_W

    @pl.when(lax.axis_index("s") == 0)
    def _():
        pltpu.sync_copy(table_hbm, table_sp)

    for s in range(NHS):
        pltpu.sync_copy(hists_hbm.at[:, pl.ds(nbase + s * HSUB, HSUB)], hsl)

        @plsc.parallel_loop(0, HSUB, step=L, unroll=2)
        def _(i):
            acc = hsl[0, pl.ds(i, L)]
            for t in range(1, NW):
                acc = acc + hsl[t, pl.ds(i, L)]
            acc = jnp.minimum(acc, MAX_DEG)
            didx[pl.ds(s * HSUB + i, L)] = acc

    plsc.subcore_barrier()

    gsems = (gsem0, gsem1)
    rbufs = (rows0, rows1)
    desc = pltpu.async_copy(
        table_sp.at[didx.at[pl.ds(0, GROWS)]], rows0, gsem0)
    for j in range(NG):
        nxt = None
        if j + 1 < NG:
            b = (j + 1) % 2
            nxt = pltpu.async_copy(
                table_sp.at[didx.at[pl.ds((j + 1) * GROWS, GROWS)]],
                rbufs[b], gsems[b])
        desc.wait()
        pltpu.sync_copy(rbufs[j % 2],
                        out_hbm.at[pl.ds(nbase + j * GROWS, GROWS)])
        desc = nxt


def kernel(edge_index, num_nodes, table):
    edges = edge_index[0].reshape(EROWS, NW * ECOLS_W)
    hists = _hist_kernel(edges)
    out = _lookup_kernel(hists, table)
    return out[:N_NODES]


# speedup vs baseline: 2.7589x; 1.1318x over previous
import functools

import jax
import jax.numpy as jnp
from jax import lax
from jax.experimental import pallas as pl
from jax.experimental.pallas import tpu as pltpu
from jax.experimental.pallas import tpu_sc as plsc

NC, NS, L = 2, 16, 16
NW = NC * NS

N_NODES = 100000
N_PAD = 102400
N_PER_W = N_PAD // NW
N_EDGES = 6400000
E_PER_W = N_EDGES // NW
CHUNK = 8000
N_CHUNKS = E_PER_W // CHUNK
GROWS = 128
NG = N_PER_W // GROWS
HSUB = 640
NHS = N_PER_W // HSUB
MAX_DEG = 256
EMB = 64

_mesh = dict(core_axis_name="c", subcore_axis_name="s")


@functools.partial(
    pl.kernel,
    out_type=jax.ShapeDtypeStruct((NW, N_PAD), jnp.int32),
    mesh=plsc.VectorSubcoreMesh(**_mesh),
    scratch_types=[
        pltpu.VMEM((CHUNK,), jnp.int32),
        pltpu.VMEM((CHUNK,), jnp.int32),
        pltpu.VMEM((N_PAD,), jnp.int32),
        pltpu.SemaphoreType.DMA,
        pltpu.SemaphoreType.DMA,
    ],
    compiler_params=pltpu.CompilerParams(needs_layout_passes=False,
                                         use_tc_tiling_on_sc=False),
)
def _hist_kernel(edges_hbm, hists_hbm, ebuf0, ebuf1, hist, sem0, sem1):
    wid = lax.axis_index("s") * NC + lax.axis_index("c")
    zeros = jnp.zeros((L,), jnp.int32)
    ones = jnp.ones((L,), jnp.int32)

    @plsc.parallel_loop(0, N_PAD, step=L, unroll=8)
    def _(i):
        hist[pl.ds(i, L)] = zeros

    base = wid * E_PER_W
    bufs = (ebuf0, ebuf1)
    sems = (sem0, sem1)
    desc = pltpu.async_copy(
        edges_hbm.at[0, pl.ds(base, CHUNK)], ebuf0, sem0)
    for c in range(N_CHUNKS):
        nxt = None
        if c + 1 < N_CHUNKS:
            b = (c + 1) % 2
            nxt = pltpu.async_copy(
                edges_hbm.at[0, pl.ds(base + (c + 1) * CHUNK, CHUNK)],
                bufs[b], sems[b])
        desc.wait()
        buf = bufs[c % 2]

        @plsc.parallel_loop(0, CHUNK, step=L, unroll=8)
        def _(i):
            ids = buf[pl.ds(i, L)]
            plsc.addupdate_scatter(hist, [ids], ones)

        desc = nxt

    pltpu.sync_copy(hist, hists_hbm.at[wid])


@functools.partial(
    pl.kernel,
    out_type=jax.ShapeDtypeStruct((N_NODES, EMB), jnp.float32),
    mesh=plsc.VectorSubcoreMesh(**_mesh),
    scratch_types=[
        pltpu.VMEM((NW, HSUB), jnp.int32),
        pltpu.VMEM((N_PER_W,), jnp.int32),
        pltpu.VMEM((GROWS, EMB), jnp.float32),
        pltpu.VMEM((GROWS, EMB), jnp.float32),
        pltpu.VMEM_SHARED((MAX_DEG + 1, EMB), jnp.float32),
        pltpu.SemaphoreType.DMA,
        pltpu.SemaphoreType.DMA,
    ],
    compiler_params=pltpu.CompilerParams(needs_layout_passes=False,
                                         use_tc_tiling_on_sc=False),
)
def _lookup_kernel(hists_hbm, table_hbm, out_hbm, hsl, didx, rows0, rows1,
                   table_sp, gsem0, gsem1):
    wid = lax.axis_index("s") * NC + lax.axis_index("c")
    nbase = wid * N_PER_W

    @pl.when(lax.axis_index("s") == 0)
    def _():
        pltpu.sync_copy(table_hbm, table_sp)

    for s in range(NHS):
        pltpu.sync_copy(hists_hbm.at[:, pl.ds(nbase + s * HSUB, HSUB)], hsl)

        @plsc.parallel_loop(0, HSUB, step=L, unroll=2)
        def _(i):
            acc = hsl[0, pl.ds(i, L)]
            for t in range(1, NW):
                acc = acc + hsl[t, pl.ds(i, L)]
            acc = jnp.minimum(acc, MAX_DEG)
            didx[pl.ds(s * HSUB + i, L)] = acc

    plsc.subcore_barrier()

    gsems = (gsem0, gsem1)
    rbufs = (rows0, rows1)

    def gather_pipeline(chunks):
        o0, n0 = chunks[0]
        desc = pltpu.async_copy(
            table_sp.at[didx.at[pl.ds(o0, n0)]], rbufs[0].at[pl.ds(0, n0)],
            gsems[0])
        for j, (oj, nj) in enumerate(chunks):
            nxt = None
            if j + 1 < len(chunks):
                b = (j + 1) % 2
                on, nn = chunks[j + 1]
                nxt = pltpu.async_copy(
                    table_sp.at[didx.at[pl.ds(on, nn)]],
                    rbufs[b].at[pl.ds(0, nn)], gsems[b])
            desc.wait()
            pltpu.sync_copy(rbufs[j % 2].at[pl.ds(0, nj)],
                            out_hbm.at[pl.ds(nbase + oj, nj)])
            desc = nxt

    full_chunks = [(j * GROWS, GROWS) for j in range(NG)]
    last_rows = N_NODES - (NW - 1) * N_PER_W
    last_chunks = ([(j * GROWS, GROWS) for j in range(last_rows // GROWS)]
                   + ([(last_rows // GROWS * GROWS, last_rows % GROWS)]
                      if last_rows % GROWS else []))

    @pl.when(wid < NW - 1)
    def _():
        gather_pipeline(full_chunks)

    @pl.when(wid == NW - 1)
    def _():
        gather_pipeline(last_chunks)


def kernel(edge_index, num_nodes, table):
    hists = _hist_kernel(edge_index)
    return _lookup_kernel(hists, table)


# speedup vs baseline: 2.9849x; 1.0819x over previous
import functools

import jax
import jax.numpy as jnp
from jax import lax
from jax.experimental import pallas as pl
from jax.experimental.pallas import tpu as pltpu
from jax.experimental.pallas import tpu_sc as plsc

NC, NS, L = 2, 16, 16
NW = NC * NS

N_NODES = 100000
N_PAD = 102400
N_PER_W = N_PAD // NW
N_EDGES = 6400000
E_PER_W = N_EDGES // NW
CHUNK = 8000
N_CHUNKS = E_PER_W // CHUNK
GROWS = 128
NG = N_PER_W // GROWS
HSUB = 640
NHS = N_PER_W // HSUB
MAX_DEG = 256
EMB = 64

_mesh = dict(core_axis_name="c", subcore_axis_name="s")


@functools.partial(
    pl.kernel,
    out_type=jax.ShapeDtypeStruct((NW, N_PAD), jnp.int32),
    mesh=plsc.VectorSubcoreMesh(**_mesh),
    scratch_types=[
        pltpu.VMEM((CHUNK,), jnp.int32),
        pltpu.VMEM((CHUNK,), jnp.int32),
        pltpu.VMEM((N_PAD,), jnp.int32),
        pltpu.SemaphoreType.DMA,
        pltpu.SemaphoreType.DMA,
    ],
    compiler_params=pltpu.CompilerParams(needs_layout_passes=False,
                                         use_tc_tiling_on_sc=False),
)
def _hist_kernel(edges_hbm, hists_hbm, ebuf0, ebuf1, hist, sem0, sem1):
    wid = lax.axis_index("s") * NC + lax.axis_index("c")
    zeros = jnp.zeros((L,), jnp.int32)
    ones = jnp.ones((L,), jnp.int32)

    @plsc.parallel_loop(0, N_PAD, step=L, unroll=8)
    def _(i):
        hist[pl.ds(i, L)] = zeros

    base = wid * E_PER_W
    bufs = (ebuf0, ebuf1)
    sems = (sem0, sem1)
    desc = pltpu.async_copy(
        edges_hbm.at[0, pl.ds(base, CHUNK)], ebuf0, sem0)
    for c in range(N_CHUNKS):
        nxt = None
        if c + 1 < N_CHUNKS:
            b = (c + 1) % 2
            nxt = pltpu.async_copy(
                edges_hbm.at[0, pl.ds(base + (c + 1) * CHUNK, CHUNK)],
                bufs[b], sems[b])
        desc.wait()
        buf = bufs[c % 2]

        @plsc.parallel_loop(0, CHUNK, step=L, unroll=8)
        def _(i):
            ids = buf[pl.ds(i, L)]
            plsc.addupdate_scatter(hist, [ids], ones)

        desc = nxt

    pltpu.sync_copy(hist, hists_hbm.at[wid])


@functools.partial(
    pl.kernel,
    out_type=jax.ShapeDtypeStruct((N_NODES, EMB), jnp.float32),
    mesh=plsc.VectorSubcoreMesh(**_mesh),
    scratch_types=[
        pltpu.VMEM((NW, HSUB), jnp.int32),
        pltpu.VMEM((N_PER_W,), jnp.int32),
        pltpu.VMEM((GROWS, EMB), jnp.float32),
        pltpu.VMEM((GROWS, EMB), jnp.float32),
        pltpu.VMEM_SHARED((MAX_DEG + 1, EMB), jnp.float32),
        pltpu.SemaphoreType.DMA,
        pltpu.SemaphoreType.DMA,
    ],
    compiler_params=pltpu.CompilerParams(needs_layout_passes=False),
)
def _lookup_kernel(hists_hbm, table_hbm, out_hbm, hsl, didx, rows0, rows1,
                   table_sp, gsem0, gsem1):
    wid = lax.axis_index("s") * NC + lax.axis_index("c")
    nbase = wid * N_PER_W

    @pl.when(lax.axis_index("s") == 0)
    def _():
        pltpu.sync_copy(table_hbm, table_sp)

    for s in range(NHS):
        pltpu.sync_copy(hists_hbm.at[:, pl.ds(nbase + s * HSUB, HSUB)], hsl)

        @plsc.parallel_loop(0, HSUB, step=L, unroll=2)
        def _(i):
            acc = hsl[0, pl.ds(i, L)]
            for t in range(1, NW):
                acc = acc + hsl[t, pl.ds(i, L)]
            acc = jnp.minimum(acc, MAX_DEG)
            didx[pl.ds(s * HSUB + i, L)] = acc

    plsc.subcore_barrier()

    gsems = (gsem0, gsem1)
    rbufs = (rows0, rows1)

    def gather_pipeline(chunks):
        o0, n0 = chunks[0]
        desc = pltpu.async_copy(
            table_sp.at[didx.at[pl.ds(o0, n0)]], rbufs[0].at[pl.ds(0, n0)],
            gsems[0])
        for j, (oj, nj) in enumerate(chunks):
            nxt = None
            if j + 1 < len(chunks):
                b = (j + 1) % 2
                on, nn = chunks[j + 1]
                nxt = pltpu.async_copy(
                    table_sp.at[didx.at[pl.ds(on, nn)]],
                    rbufs[b].at[pl.ds(0, nn)], gsems[b])
            desc.wait()
            pltpu.sync_copy(rbufs[j % 2].at[pl.ds(0, nj)],
                            out_hbm.at[pl.ds(nbase + oj, nj)])
            desc = nxt

    full_chunks = [(j * GROWS, GROWS) for j in range(NG)]
    last_rows = N_NODES - (NW - 1) * N_PER_W
    last_chunks = ([(j * GROWS, GROWS) for j in range(last_rows // GROWS)]
                   + ([(last_rows // GROWS * GROWS, last_rows % GROWS)]
                      if last_rows % GROWS else []))

    @pl.when(wid < NW - 1)
    def _():
        gather_pipeline(full_chunks)

    @pl.when(wid == NW - 1)
    def _():
        gather_pipeline(last_chunks)


def kernel(edge_index, num_nodes, table):
    hists = _hist_kernel(edge_index)
    return _lookup_kernel(hists, table)


# speedup vs baseline: 3.0712x; 1.0289x over previous
import functools

import jax
import jax.numpy as jnp
from jax import lax
from jax.experimental import pallas as pl
from jax.experimental.pallas import tpu as pltpu
from jax.experimental.pallas import tpu_sc as plsc

NC, NS, L = 2, 16, 16
NW = NC * NS

N_NODES = 100000
N_PAD = 102400
N_PER_W = N_PAD // NW
N_EDGES = 6400000
E_PER_W = N_EDGES // NW
CHUNK = 8000
N_CHUNKS = E_PER_W // CHUNK
GROWS = 128
NG = N_PER_W // GROWS
HSUB = 640
NHS = N_PER_W // HSUB
MAX_DEG = 256
EMB = 64

_mesh = dict(core_axis_name="c", subcore_axis_name="s")


@functools.partial(
    pl.kernel,
    out_type=jax.ShapeDtypeStruct((NW, N_PAD), jnp.int32),
    mesh=plsc.VectorSubcoreMesh(**_mesh),
    scratch_types=[
        pltpu.VMEM((CHUNK,), jnp.int32),
        pltpu.VMEM((CHUNK,), jnp.int32),
        pltpu.VMEM((N_PAD,), jnp.int32),
        pltpu.SemaphoreType.DMA,
        pltpu.SemaphoreType.DMA,
    ],
    compiler_params=pltpu.CompilerParams(needs_layout_passes=False,
                                         use_tc_tiling_on_sc=False),
)
def _hist_kernel(edges_hbm, hists_hbm, ebuf0, ebuf1, hist, sem0, sem1):
    wid = lax.axis_index("s") * NC + lax.axis_index("c")
    zeros = jnp.zeros((L,), jnp.int32)
    ones = jnp.ones((L,), jnp.int32)

    @plsc.parallel_loop(0, N_PAD, step=L, unroll=8)
    def _(i):
        hist[pl.ds(i, L)] = zeros

    base = wid * E_PER_W
    bufs = (ebuf0, ebuf1)
    sems = (sem0, sem1)
    desc = pltpu.async_copy(
        edges_hbm.at[pl.ds(base, CHUNK)], ebuf0, sem0)
    for c in range(N_CHUNKS):
        nxt = None
        if c + 1 < N_CHUNKS:
            b = (c + 1) % 2
            nxt = pltpu.async_copy(
                edges_hbm.at[pl.ds(base + (c + 1) * CHUNK, CHUNK)],
                bufs[b], sems[b])
        desc.wait()
        buf = bufs[c % 2]

        @plsc.parallel_loop(0, CHUNK, step=L, unroll=8)
        def _(i):
            ids = buf[pl.ds(i, L)]
            plsc.addupdate_scatter(hist, [ids], ones)

        desc = nxt

    pltpu.sync_copy(hist, hists_hbm.at[wid])


@functools.partial(
    pl.kernel,
    out_type=jax.ShapeDtypeStruct((N_NODES, EMB), jnp.float32),
    mesh=plsc.VectorSubcoreMesh(**_mesh),
    scratch_types=[
        pltpu.VMEM((NW, HSUB), jnp.int32),
        pltpu.VMEM((N_PER_W,), jnp.int32),
        pltpu.VMEM((GROWS, EMB), jnp.float32),
        pltpu.VMEM((GROWS, EMB), jnp.float32),
        pltpu.VMEM_SHARED((MAX_DEG + 1, EMB), jnp.float32),
        pltpu.SemaphoreType.DMA,
        pltpu.SemaphoreType.DMA,
    ],
    compiler_params=pltpu.CompilerParams(needs_layout_passes=False,
                                         use_tc_tiling_on_sc=False),
)
def _lookup_kernel(hists_hbm, table_hbm, out_hbm, hsl, didx, rows0, rows1,
                   table_sp, gsem0, gsem1):
    wid = lax.axis_index("s") * NC + lax.axis_index("c")
    nbase = wid * N_PER_W

    @pl.when(lax.axis_index("s") == 0)
    def _():
        pltpu.sync_copy(table_hbm, table_sp)

    for s in range(NHS):
        pltpu.sync_copy(hists_hbm.at[:, pl.ds(nbase + s * HSUB, HSUB)], hsl)

        @plsc.parallel_loop(0, HSUB, step=L, unroll=2)
        def _(i):
            acc = hsl[0, pl.ds(i, L)]
            for t in range(1, NW):
                acc = acc + hsl[t, pl.ds(i, L)]
            acc = jnp.minimum(acc, MAX_DEG)
            didx[pl.ds(s * HSUB + i, L)] = acc

    plsc.subcore_barrier()

    gsems = (gsem0, gsem1)
    rbufs = (rows0, rows1)

    def gather_pipeline(chunks):
        o0, n0 = chunks[0]
        desc = pltpu.async_copy(
            table_sp.at[didx.at[pl.ds(o0, n0)]], rbufs[0].at[pl.ds(0, n0)],
            gsems[0])
        for j, (oj, nj) in enumerate(chunks):
            nxt = None
            if j + 1 < len(chunks):
                b = (j + 1) % 2
                on, nn = chunks[j + 1]
                nxt = pltpu.async_copy(
                    table_sp.at[didx.at[pl.ds(on, nn)]],
                    rbufs[b].at[pl.ds(0, nn)], gsems[b])
            desc.wait()
            pltpu.sync_copy(rbufs[j % 2].at[pl.ds(0, nj)],
                            out_hbm.at[pl.ds(nbase + oj, nj)])
            desc = nxt

    full_chunks = [(j * GROWS, GROWS) for j in range(NG)]
    last_rows = N_NODES - (NW - 1) * N_PER_W
    last_chunks = ([(j * GROWS, GROWS) for j in range(last_rows // GROWS)]
                   + ([(last_rows // GROWS * GROWS, last_rows % GROWS)]
                      if last_rows % GROWS else []))

    @pl.when(wid < NW - 1)
    def _():
        gather_pipeline(full_chunks)

    @pl.when(wid == NW - 1)
    def _():
        gather_pipeline(last_chunks)


def kernel(edge_index, num_nodes, table):
    hists = _hist_kernel(edge_index[0])
    return _lookup_kernel(hists, table)
